# Initial kernel scaffold; baseline (speedup 1.0000x reference)
import jax
import jax.numpy as jnp
from jax.experimental import pallas as pl


def kernel(x, table):
    raise NotImplementedError("write your pallas kernel here")


# speedup vs baseline: 1.4802x; 1.4802x over previous
import functools

import jax
import jax.numpy as jnp
from jax import lax
from jax.experimental import pallas as pl
from jax.experimental.pallas import tpu as pltpu
from jax.experimental.pallas import tpu_sc as plsc


def _gather_call(n_total, v, d, n_workers, chunk):
    n_chunks = (n_total // n_workers) // chunk
    b_per_w = n_total // n_workers
    mesh = plsc.VectorSubcoreMesh(core_axis_name="c", subcore_axis_name="s")

    @functools.partial(
        pl.kernel,
        mesh=mesh,
        out_type=jax.ShapeDtypeStruct((n_total, d), jnp.float32),
        compiler_params=pltpu.CompilerParams(use_tc_tiling_on_sc=False),
        scratch_types=[
            pltpu.VMEM((chunk,), jnp.int32),
            pltpu.VMEM((chunk, d), jnp.float32),
            pltpu.SemaphoreType.DMA,
        ],
    )
    def gather_kernel(idx_hbm, table_hbm, out_hbm, idx_v, rows_v, sem):
        wid = lax.axis_index("s") * 2 + lax.axis_index("c")
        base = pl.multiple_of(wid * b_per_w, 8)

        def body(i, carry):
            start = pl.multiple_of(base + i * chunk, 8)
            pltpu.sync_copy(idx_hbm.at[pl.ds(start, chunk)], idx_v)
            pltpu.async_copy(table_hbm.at[idx_v], rows_v, sem).wait()
            pltpu.sync_copy(rows_v, out_hbm.at[pl.ds(start, chunk)])
            return carry

        lax.fori_loop(0, n_chunks, body, 0)

    return gather_kernel


def kernel(x, table):
    b, h = x.shape
    v, d = table.shape
    n = b * h
    n_workers = 32
    chunk = 1600
    out = _gather_call(n, v, d, n_workers, chunk)(x.reshape(n), table)
    return out.reshape(b, h, d)


# speedup vs baseline: 1.4915x; 1.0076x over previous
import functools

import jax
import jax.numpy as jnp
from jax import lax
from jax.experimental import pallas as pl
from jax.experimental.pallas import tpu as pltpu
from jax.experimental.pallas import tpu_sc as plsc


def _gather_call(n_total, d, n_workers, chunk):
    b_per_w = n_total // n_workers
    n_chunks = b_per_w // chunk
    mesh = plsc.VectorSubcoreMesh(core_axis_name="c", subcore_axis_name="s")

    @functools.partial(
        pl.kernel,
        mesh=mesh,
        out_type=jax.ShapeDtypeStruct((n_total, d), jnp.float32),
        compiler_params=pltpu.CompilerParams(use_tc_tiling_on_sc=False),
        scratch_types=[
            pltpu.VMEM((2, chunk), jnp.int32),
            pltpu.VMEM((2, chunk, d), jnp.float32),
            pltpu.SemaphoreType.DMA,
            pltpu.SemaphoreType.DMA,
            pltpu.SemaphoreType.DMA,
            pltpu.SemaphoreType.DMA,
            pltpu.SemaphoreType.DMA,
            pltpu.SemaphoreType.DMA,
        ],
    )
    def gather_kernel(idx_hbm, table_hbm, out_hbm, idx_v, rows_v, i0, i1,
                      g0, g1, s0, s1):
        wid = lax.axis_index("s") * 2 + lax.axis_index("c")
        base = pl.multiple_of(wid * b_per_w, 8)
        isem = (i0, i1)
        gsem = (g0, g1)
        ssem = (s0, s1)

        def start(i):
            return pl.multiple_of(base + i * chunk, 8)

        idx_loads = [None] * n_chunks
        scatters = [None] * n_chunks
        idx_loads[0] = pltpu.async_copy(
            idx_hbm.at[pl.ds(start(0), chunk)], idx_v.at[0], isem[0])
        for i in range(n_chunks):
            b = i % 2
            if i >= 2:
                scatters[i - 2].wait()
            idx_loads[i].wait()
            gather = pltpu.async_copy(
                table_hbm.at[idx_v.at[b]], rows_v.at[b], gsem[b])
            if i + 1 < n_chunks:
                nb = (i + 1) % 2
                idx_loads[i + 1] = pltpu.async_copy(
                    idx_hbm.at[pl.ds(start(i + 1), chunk)], idx_v.at[nb],
                    isem[nb])
            gather.wait()
            scatters[i] = pltpu.async_copy(
                rows_v.at[b], out_hbm.at[pl.ds(start(i), chunk)], ssem[b])
        scatters[n_chunks - 2].wait()
        scatters[n_chunks - 1].wait()

    return gather_kernel


def kernel(x, table):
    b, h = x.shape
    v, d = table.shape
    n = b * h
    out = _gather_call(n, d, n_workers=32, chunk=1600)(x.reshape(n), table)
    return out.reshape(b, h, d)
